# baseline (speedup 1.0000x reference)
import functools

import jax
import jax.numpy as jnp
from jax import lax
from jax.experimental import pallas as pl
from jax.experimental.pallas import tpu as pltpu
from jax.experimental.pallas import tpu_sc as plsc

_H = 640
_B = 16
_NPIX = _H * _H
_WROW = _H // 8
_TROW = _WROW + 1
_PWORDS = _H * _TROW
_HROWS = _H // 2
_NSLABS = _HROWS // 8
_NW = 32


def _tile_body(dist_h, gdist_h, gi_h, tm_h, gk_h, out_h,
               tbl,
               a_d0, a_d1, a_g0, a_g1, a_gi, a_tm,
               c_d0, c_d1, c_g0, c_g1, c_gi, c_tm,
               res, pk_hbm, semA, semB):
    wid = lax.axis_index("c") * 16 + lax.axis_index("s")
    b = wid // 2
    half = wid % 2
    r0 = half * _HROWS
    lanes = lax.iota(jnp.int32, 16)
    tb0 = half * (_PWORDS // 2)
    bufsA = (a_d0, a_d1, a_g0, a_g1, a_gi, a_tm)
    bufsB = (c_d0, c_d1, c_g0, c_g1, c_gi, c_tm)

    def gk_issue(sl, buf, sem):
        pltpu.async_copy(gk_h.at[b, pl.ds(r0 + sl * 8, 8), :], buf, sem)

    def gk_drain(sl, buf, sem):
        pltpu.make_async_copy(gk_h.at[b, pl.ds(r0 + sl * 8, 8), :], buf,
                              sem).wait()

    def pack_slab(sl, buf):
        @plsc.parallel_loop(0, 8)
        def pack_row(rr):
            for t in range(_WROW // 16):
                c0 = t * 16
                w = buf[rr, pl.ds(c0, 16)]
                for j in range(1, 8):
                    w = w | (buf[rr, pl.ds(j * _WROW + c0, 16)] << (4 * j))
                tbl[pl.ds(tb0 + (sl * 8 + rr) * _TROW + c0, 16)] = w

    gk_issue(0, a_gi, semA)

    def pack_pair(k, _):
        sl0 = 2 * k
        gk_issue(sl0 + 1, c_gi, semB)
        gk_drain(sl0, a_gi, semA)
        pack_slab(sl0, a_gi)
        gk_issue(lax.rem(sl0 + 2, _NSLABS), a_gi, semA)
        gk_drain(sl0 + 1, c_gi, semB)
        pack_slab(sl0 + 1, c_gi)
        return 0

    lax.fori_loop(0, _NSLABS // 2, pack_pair, 0)
    gk_drain(0, a_gi, semA)
    pltpu.sync_copy(tbl.at[pl.ds(tb0, _PWORDS // 2)],
                    pk_hbm.at[pl.ds(b * _PWORDS + tb0, _PWORDS // 2)])
    plsc.subcore_barrier()
    ob0 = (1 - half) * (_PWORDS // 2)
    pltpu.sync_copy(pk_hbm.at[pl.ds(b * _PWORDS + ob0, _PWORDS // 2)],
                    tbl.at[pl.ds(ob0, _PWORDS // 2)])

    def slab_srcs(sl):
        rbase = r0 + sl * 8
        return (dist_h.at[b, 0, pl.ds(rbase, 8), :],
                dist_h.at[b, 1, pl.ds(rbase, 8), :],
                gdist_h.at[b, 0, pl.ds(rbase, 8), :],
                gdist_h.at[b, 1, pl.ds(rbase, 8), :],
                gi_h.at[b, pl.ds(rbase, 8), :],
                tm_h.at[b, pl.ds(rbase, 8), :])

    def issue(sl, bufs, sem):
        for src, dst in zip(slab_srcs(sl), bufs):
            pltpu.async_copy(src, dst, sem)

    def drain(sl, bufs, sem):
        for src, dst in zip(slab_srcs(sl), bufs):
            pltpu.make_async_copy(src, dst, sem).wait()

    def compute(sl, bufs, accs):
        d0b, d1b, g0b, g1b, gib, tmb = bufs
        rbase = r0 + sl * 8

        def row_body(rr, accs2):
            y_f = jnp.full((16,), rbase + rr, jnp.int32).astype(jnp.float32)

            @plsc.parallel_loop(0, _H // 16, unroll=8, carry=accs2)
            def vec_body(t, accs3):
                al, am, at_ = accs3
                c0 = t * 16
                c_f = (c0 + lanes).astype(jnp.float32)
                d0v = d0b[rr, pl.ds(c0, 16)]
                d1v = d1b[rr, pl.ds(c0, 16)]
                offx = jnp.clip((c_f + 10.0 * d0v).astype(jnp.int32),
                                0, _H - 1)
                offy = jnp.clip((y_f + 10.0 * d1v).astype(jnp.int32),
                                0, _H - 1)
                nib = offx // _WROW
                wx = offx - nib * _WROW
                word = plsc.load_gather(tbl, [offy * _TROW + wx])
                val = lax.shift_right_logical(word, nib * 4) & 0xF
                giv = gib[rr, pl.ds(c0, 16)]
                tmv = tmb[rr, pl.ds(c0, 16)]
                tmf = tmv.astype(jnp.float32)
                m = jnp.where(giv != val, tmf, 0.0)
                g0v = g0b[rr, pl.ds(c0, 16)]
                g1v = g1b[rr, pl.ds(c0, 16)]
                ad0 = jnp.abs(d0v - g0v)
                ad1 = jnp.abs(d1v - g1v)
                h0 = jnp.minimum(ad0, 1.0)
                h1 = jnp.minimum(ad1, 1.0)
                l0 = h0 * (ad0 - 0.5 * h0)
                l1 = h1 * (ad1 - 0.5 * h1)
                return (al + m * (l0 + l1), am + m, at_ + tmf)

            return vec_body

        return lax.fori_loop(0, 8, row_body, accs)

    issue(0, bufsA, semA)

    def pair_body(k, accs):
        sl0 = 2 * k
        issue(sl0 + 1, bufsB, semB)
        drain(sl0, bufsA, semA)
        accs = compute(sl0, bufsA, accs)
        issue(lax.rem(sl0 + 2, _NSLABS), bufsA, semA)
        drain(sl0 + 1, bufsB, semB)
        return compute(sl0 + 1, bufsB, accs)

    zero = jnp.zeros((16,), jnp.float32)
    a_loss, a_msk, a_tm = lax.fori_loop(0, _NSLABS // 2, pair_body,
                                        (zero, zero, zero))
    drain(0, bufsA, semA)
    res[pl.ds(0, 16)] = a_loss
    res[pl.ds(16, 16)] = a_msk
    res[pl.ds(32, 16)] = a_tm
    pltpu.sync_copy(res, out_h.at[pl.ds(wid * 48, 48)])


@jax.jit
def kernel(distances, gt_instances, gt_kernel_instances, training_masks, gt_distances):
    eps = 1e-6
    mesh = plsc.VectorSubcoreMesh(core_axis_name="c", subcore_axis_name="s")
    dense = [pltpu.VMEM((8, _H), jnp.float32)] * 4 + [pltpu.VMEM((8, _H), jnp.int32)] * 2
    run = pl.kernel(
        _tile_body,
        out_type=jax.ShapeDtypeStruct((_NW * 48,), jnp.float32),
        mesh=mesh,
        compiler_params=pltpu.CompilerParams(
            needs_layout_passes=False, use_tc_tiling_on_sc=True),
        scratch_types=(
            [pltpu.VMEM((_PWORDS,), jnp.int32)]
            + dense + dense
            + [pltpu.VMEM((48,), jnp.float32),
               pltpu.HBM((_B * _PWORDS,), jnp.int32),
               pltpu.SemaphoreType.DMA,
               pltpu.SemaphoreType.DMA]
        ),
    )
    out = run(distances, gt_distances, gt_instances, training_masks,
              gt_kernel_instances)
    sums = out.reshape(_B, 2, 3, 16).sum(axis=(1, 3))
    loss_sum, mask_sum, tm_sum = sums[:, 0], sums[:, 1], sums[:, 2]
    loss = jnp.mean(loss_sum / (mask_sum + eps))
    iou_text = (tm_sum - mask_sum) / (tm_sum + eps)
    return loss, iou_text


# speedup vs baseline: 1.0328x; 1.0328x over previous
import jax
import jax.numpy as jnp
from jax import lax
from jax.experimental import pallas as pl
from jax.experimental.pallas import tpu as pltpu
from jax.experimental.pallas import tpu_sc as plsc

_H = 640
_B = 16
_NPIX = _H * _H
_WROW = _H // 8
_TROW = _WROW
_PWORDS = _H * _TROW
_HROWS = _H // 2
_NSLABS = _HROWS // 8
_NW = 32


def _tile_body(dist_h, gdist_h, gi_h, tm_h, gk_h, out_h,
               tbl,
               a_d0, a_d1, a_g0, a_g1, a_gi, a_tm,
               c_d0, c_d1, c_g0, c_g1, c_gi, c_tm,
               res, pk_hbm, semA, semB):
    wid = lax.axis_index("c") * 16 + lax.axis_index("s")
    b = wid // 2
    half = wid % 2
    r0 = half * _HROWS
    lanes = lax.iota(jnp.int32, 16)
    tb0 = half * (_PWORDS // 2)
    bufsA = (a_d0, a_d1, a_g0, a_g1, a_gi, a_tm)
    bufsB = (c_d0, c_d1, c_g0, c_g1, c_gi, c_tm)

    def gk_issue(sl, buf, sem):
        pltpu.async_copy(gk_h.at[b, pl.ds(r0 + sl * 8, 8), :], buf, sem)

    def gk_drain(sl, buf, sem):
        pltpu.make_async_copy(gk_h.at[b, pl.ds(r0 + sl * 8, 8), :], buf,
                              sem).wait()

    def pack_slab(sl, buf):
        @plsc.parallel_loop(0, 8)
        def pack_row(rr):
            for t in range(_WROW // 16):
                c0 = t * 16
                w = buf[rr, pl.ds(c0, 16)]
                for j in range(1, 8):
                    w = w | (buf[rr, pl.ds(j * _WROW + c0, 16)] << (4 * j))
                tbl[pl.ds(tb0 + (sl * 8 + rr) * _TROW + c0, 16)] = w

    gk_issue(0, a_gi, semA)

    def pack_pair(k, _):
        sl0 = 2 * k
        gk_issue(sl0 + 1, c_gi, semB)
        gk_drain(sl0, a_gi, semA)
        pack_slab(sl0, a_gi)
        gk_issue(lax.rem(sl0 + 2, _NSLABS), a_gi, semA)
        gk_drain(sl0 + 1, c_gi, semB)
        pack_slab(sl0 + 1, c_gi)
        return 0

    lax.fori_loop(0, _NSLABS // 2, pack_pair, 0)
    gk_drain(0, a_gi, semA)
    pltpu.sync_copy(tbl.at[pl.ds(tb0, _PWORDS // 2)],
                    pk_hbm.at[pl.ds(b * _PWORDS + tb0, _PWORDS // 2)])
    plsc.subcore_barrier()
    ob0 = (1 - half) * (_PWORDS // 2)
    pltpu.sync_copy(pk_hbm.at[pl.ds(b * _PWORDS + ob0, _PWORDS // 2)],
                    tbl.at[pl.ds(ob0, _PWORDS // 2)])

    def slab_srcs(sl):
        rbase = r0 + sl * 8
        return (dist_h.at[b, 0, pl.ds(rbase, 8), :],
                dist_h.at[b, 1, pl.ds(rbase, 8), :],
                gdist_h.at[b, 0, pl.ds(rbase, 8), :],
                gdist_h.at[b, 1, pl.ds(rbase, 8), :],
                gi_h.at[b, pl.ds(rbase, 8), :],
                tm_h.at[b, pl.ds(rbase, 8), :])

    def issue(sl, bufs, sem):
        for src, dst in zip(slab_srcs(sl), bufs):
            pltpu.async_copy(src, dst, sem)

    def drain(sl, bufs, sem):
        for src, dst in zip(slab_srcs(sl), bufs):
            pltpu.make_async_copy(src, dst, sem).wait()

    def compute(sl, bufs, accs):
        d0b, d1b, g0b, g1b, gib, tmb = bufs
        rbase = r0 + sl * 8

        def row_body(rr, accs2):
            y_f = jnp.full((16,), rbase + rr, jnp.int32).astype(jnp.float32)

            @plsc.parallel_loop(0, _H // 16, unroll=8, carry=accs2)
            def vec_body(t, accs3):
                al, am, at_ = accs3
                c0 = t * 16
                c_f = (c0 + lanes).astype(jnp.float32)
                d0v = d0b[rr, pl.ds(c0, 16)]
                d1v = d1b[rr, pl.ds(c0, 16)]
                offx = jnp.clip((c_f + 10.0 * d0v).astype(jnp.int32),
                                0, _H - 1)
                offy = jnp.clip((y_f + 10.0 * d1v).astype(jnp.int32),
                                0, _H - 1)
                nib = offx // _WROW
                wx = offx - nib * _WROW
                word = plsc.load_gather(tbl, [offy * _TROW + wx])
                val = lax.shift_right_logical(word, nib * 4) & 0xF
                giv = gib[rr, pl.ds(c0, 16)]
                tmv = tmb[rr, pl.ds(c0, 16)]
                tmf = tmv.astype(jnp.float32)
                m = jnp.where(giv != val, tmf, 0.0)
                g0v = g0b[rr, pl.ds(c0, 16)]
                g1v = g1b[rr, pl.ds(c0, 16)]
                diff0 = jnp.abs(d0v - g0v) * m
                diff1 = jnp.abs(d1v - g1v) * m
                m10 = jnp.minimum(diff0, 1.0)
                m11 = jnp.minimum(diff1, 1.0)
                l0 = m10 * (diff0 - 0.5 * m10)
                l1 = m11 * (diff1 - 0.5 * m11)
                return (al + (l0 + l1), am + m, at_ + tmf)

            return vec_body

        return lax.fori_loop(0, 8, row_body, accs)

    issue(0, bufsA, semA)

    def pair_body(k, accs):
        sl0 = 2 * k
        issue(sl0 + 1, bufsB, semB)
        drain(sl0, bufsA, semA)
        accs = compute(sl0, bufsA, accs)
        issue(lax.rem(sl0 + 2, _NSLABS), bufsA, semA)
        drain(sl0 + 1, bufsB, semB)
        return compute(sl0 + 1, bufsB, accs)

    zero = jnp.zeros((16,), jnp.float32)
    a_loss, a_msk, a_tm = lax.fori_loop(0, _NSLABS // 2, pair_body,
                                        (zero, zero, zero))
    drain(0, bufsA, semA)
    res[pl.ds(0, 16)] = a_loss
    res[pl.ds(16, 16)] = a_msk
    res[pl.ds(32, 16)] = a_tm
    pltpu.sync_copy(res, out_h.at[pl.ds(wid * 48, 48)])


@jax.jit
def kernel(distances, gt_instances, gt_kernel_instances, training_masks, gt_distances):
    eps = 1e-6
    mesh = plsc.VectorSubcoreMesh(core_axis_name="c", subcore_axis_name="s")
    dense = [pltpu.VMEM((8, _H), jnp.float32)] * 4 + [pltpu.VMEM((8, _H), jnp.int32)] * 2
    run = pl.kernel(
        _tile_body,
        out_type=jax.ShapeDtypeStruct((_NW * 48,), jnp.float32),
        mesh=mesh,
        compiler_params=pltpu.CompilerParams(
            needs_layout_passes=False, use_tc_tiling_on_sc=True),
        scratch_types=(
            [pltpu.VMEM((_PWORDS,), jnp.int32)]
            + dense + dense
            + [pltpu.VMEM((48,), jnp.float32),
               pltpu.HBM((_B * _PWORDS,), jnp.int32),
               pltpu.SemaphoreType.DMA,
               pltpu.SemaphoreType.DMA]
        ),
    )
    out = run(distances, gt_distances, gt_instances, training_masks,
              gt_kernel_instances)
    sums = out.reshape(_B, 2, 3, 16).sum(axis=(1, 3))
    loss_sum, mask_sum, tm_sum = sums[:, 0], sums[:, 1], sums[:, 2]
    loss = jnp.mean(loss_sum / (mask_sum + eps))
    iou_text = (tm_sum - mask_sum) / (tm_sum + eps)
    return loss, iou_text


# speedup vs baseline: 1.0480x; 1.0147x over previous
import jax
import jax.numpy as jnp
from jax import lax
from jax.experimental import pallas as pl
from jax.experimental.pallas import tpu as pltpu
from jax.experimental.pallas import tpu_sc as plsc

_H = 640
_B = 16
_NPIX = _H * _H
_WROW = _H // 8
_TROW = _WROW
_PWORDS = _H * _TROW
_HROWS = _H // 2
_NSLABS = _HROWS // 8
_NW = 32


def _tile_body(dist_h, gdist_h, gi_h, tm_h, gk_h, out_h,
               tbl,
               a_d0, a_d1, a_g0, a_g1, a_gi, a_tm,
               c_d0, c_d1, c_g0, c_g1, c_gi, c_tm,
               res, pk_hbm, semA, semB):
    wid = lax.axis_index("c") * 16 + lax.axis_index("s")
    b = wid // 2
    half = wid % 2
    r0 = half * _HROWS
    lanes = lax.iota(jnp.int32, 16)
    tb0 = half * (_PWORDS // 2)
    bufsA = (a_d0, a_d1, a_g0, a_g1, a_gi, a_tm)
    bufsB = (c_d0, c_d1, c_g0, c_g1, c_gi, c_tm)

    def gk_issue(sl, buf, sem):
        pltpu.async_copy(gk_h.at[b, pl.ds(r0 + sl * 8, 8), :], buf, sem)

    def gk_drain(sl, buf, sem):
        pltpu.make_async_copy(gk_h.at[b, pl.ds(r0 + sl * 8, 8), :], buf,
                              sem).wait()

    def pack_slab(sl, buf):
        @plsc.parallel_loop(0, 8)
        def pack_row(rr):
            for t in range(_WROW // 16):
                c0 = t * 16
                w = buf[rr, pl.ds(c0, 16)]
                for j in range(1, 8):
                    w = w | (buf[rr, pl.ds(j * _WROW + c0, 16)] << (4 * j))
                tbl[pl.ds(tb0 + (sl * 8 + rr) * _TROW + c0, 16)] = w

    gk_issue(0, a_gi, semA)

    def pack_pair(k, _):
        sl0 = 2 * k
        gk_issue(sl0 + 1, c_gi, semB)
        gk_drain(sl0, a_gi, semA)
        pack_slab(sl0, a_gi)
        gk_issue(lax.rem(sl0 + 2, _NSLABS), a_gi, semA)
        gk_drain(sl0 + 1, c_gi, semB)
        pack_slab(sl0 + 1, c_gi)
        return 0

    lax.fori_loop(0, _NSLABS // 2, pack_pair, 0)
    gk_drain(0, a_gi, semA)

    def slab_srcs(sl):
        rbase = r0 + sl * 8
        return (dist_h.at[b, 0, pl.ds(rbase, 8), :],
                dist_h.at[b, 1, pl.ds(rbase, 8), :],
                gdist_h.at[b, 0, pl.ds(rbase, 8), :],
                gdist_h.at[b, 1, pl.ds(rbase, 8), :],
                gi_h.at[b, pl.ds(rbase, 8), :],
                tm_h.at[b, pl.ds(rbase, 8), :])

    def issue(sl, bufs, sem):
        for src, dst in zip(slab_srcs(sl), bufs):
            pltpu.async_copy(src, dst, sem)

    def drain(sl, bufs, sem):
        for src, dst in zip(slab_srcs(sl), bufs):
            pltpu.make_async_copy(src, dst, sem).wait()

    def compute(sl, bufs, accs):
        d0b, d1b, g0b, g1b, gib, tmb = bufs
        rbase = r0 + sl * 8

        def row_body(rr, accs2):
            y_f = jnp.full((16,), rbase + rr, jnp.int32).astype(jnp.float32)

            @plsc.parallel_loop(0, _H // 16, unroll=8, carry=accs2)
            def vec_body(t, accs3):
                al, am, at_ = accs3
                c0 = t * 16
                c_f = (c0 + lanes).astype(jnp.float32)
                d0v = d0b[rr, pl.ds(c0, 16)]
                d1v = d1b[rr, pl.ds(c0, 16)]
                offx = jnp.clip((c_f + 10.0 * d0v).astype(jnp.int32),
                                0, _H - 1)
                offy = jnp.clip((y_f + 10.0 * d1v).astype(jnp.int32),
                                0, _H - 1)
                nib = offx // _WROW
                wx = offx - nib * _WROW
                word = plsc.load_gather(tbl, [offy * _TROW + wx])
                val = lax.shift_right_logical(word, nib * 4) & 0xF
                giv = gib[rr, pl.ds(c0, 16)]
                tmv = tmb[rr, pl.ds(c0, 16)]
                tmf = tmv.astype(jnp.float32)
                m = jnp.where(giv != val, tmf, 0.0)
                g0v = g0b[rr, pl.ds(c0, 16)]
                g1v = g1b[rr, pl.ds(c0, 16)]
                diff0 = jnp.abs(d0v - g0v) * m
                diff1 = jnp.abs(d1v - g1v) * m
                m10 = jnp.minimum(diff0, 1.0)
                m11 = jnp.minimum(diff1, 1.0)
                l0 = m10 * (diff0 - 0.5 * m10)
                l1 = m11 * (diff1 - 0.5 * m11)
                return (al + (l0 + l1), am + m, at_ + tmf)

            return vec_body

        return lax.fori_loop(0, 8, row_body, accs)

    issue(0, bufsA, semA)
    pltpu.sync_copy(tbl.at[pl.ds(tb0, _PWORDS // 2)],
                    pk_hbm.at[pl.ds(b * _PWORDS + tb0, _PWORDS // 2)])
    plsc.subcore_barrier()
    ob0 = (1 - half) * (_PWORDS // 2)
    pltpu.sync_copy(pk_hbm.at[pl.ds(b * _PWORDS + ob0, _PWORDS // 2)],
                    tbl.at[pl.ds(ob0, _PWORDS // 2)])

    def pair_body(k, accs):
        sl0 = 2 * k
        issue(sl0 + 1, bufsB, semB)
        drain(sl0, bufsA, semA)
        accs = compute(sl0, bufsA, accs)
        issue(lax.rem(sl0 + 2, _NSLABS), bufsA, semA)
        drain(sl0 + 1, bufsB, semB)
        return compute(sl0 + 1, bufsB, accs)

    zero = jnp.zeros((16,), jnp.float32)
    a_loss, a_msk, a_tm = lax.fori_loop(0, _NSLABS // 2, pair_body,
                                        (zero, zero, zero))
    drain(0, bufsA, semA)
    res[pl.ds(0, 16)] = a_loss
    res[pl.ds(16, 16)] = a_msk
    res[pl.ds(32, 16)] = a_tm
    pltpu.sync_copy(res, out_h.at[pl.ds(wid * 48, 48)])


@jax.jit
def kernel(distances, gt_instances, gt_kernel_instances, training_masks, gt_distances):
    eps = 1e-6
    mesh = plsc.VectorSubcoreMesh(core_axis_name="c", subcore_axis_name="s")
    dense = [pltpu.VMEM((8, _H), jnp.float32)] * 4 + [pltpu.VMEM((8, _H), jnp.int32)] * 2
    run = pl.kernel(
        _tile_body,
        out_type=jax.ShapeDtypeStruct((_NW * 48,), jnp.float32),
        mesh=mesh,
        compiler_params=pltpu.CompilerParams(
            needs_layout_passes=False, use_tc_tiling_on_sc=True),
        scratch_types=(
            [pltpu.VMEM((_PWORDS,), jnp.int32)]
            + dense + dense
            + [pltpu.VMEM((48,), jnp.float32),
               pltpu.HBM((_B * _PWORDS,), jnp.int32),
               pltpu.SemaphoreType.DMA,
               pltpu.SemaphoreType.DMA]
        ),
    )
    out = run(distances, gt_distances, gt_instances, training_masks,
              gt_kernel_instances)
    sums = out.reshape(_B, 2, 3, 16).sum(axis=(1, 3))
    loss_sum, mask_sum, tm_sum = sums[:, 0], sums[:, 1], sums[:, 2]
    loss = jnp.mean(loss_sum / (mask_sum + eps))
    iou_text = (tm_sum - mask_sum) / (tm_sum + eps)
    return loss, iou_text
